# Initial kernel scaffold; baseline (speedup 1.0000x reference)
import jax
import jax.numpy as jnp
from jax.experimental import pallas as pl


def kernel(param, grad, exp_avg, exp_avg_sq, index):
    raise NotImplementedError("write your pallas kernel here")


# speedup vs baseline: 2.5064x; 2.5064x over previous
import functools
import math

import jax
import jax.numpy as jnp
from jax import lax
from jax.experimental import pallas as pl
from jax.experimental.pallas import tpu as pltpu
from jax.experimental.pallas import tpu_sc as plsc

NC, NS, L = 2, 16, 16
NW = NC * NS

BETA1, BETA2 = 0.9, 0.999
EPS = 1e-15
LR = 1e-3
STEP = 1
_BC1 = 1.0 - BETA1 ** STEP
_BC2 = 1.0 - BETA2 ** STEP
STEP_SIZE = LR / _BC1
INV_SQRT_BC2 = 1.0 / math.sqrt(_BC2)

CHUNK = 128


def _adam_vec(p, ea, eas, g):
    ea2 = ea * BETA1 + g * (1.0 - BETA1)
    eas2 = eas * BETA2 + (g * g) * (1.0 - BETA2)
    bits = lax.bitcast_convert_type(eas2, jnp.int32)
    y = lax.bitcast_convert_type(jnp.int32(0x5F3759DF) - (bits >> 1),
                                 jnp.float32)
    xh = eas2 * 0.5
    y = y * (1.5 - xh * y * y)
    y = y * (1.5 - xh * y * y)
    y = y * (1.5 - xh * y * y)
    root = eas2 * y
    denom = root * INV_SQRT_BC2 + EPS
    p2 = p - STEP_SIZE * (ea2 / denom)
    return p2, ea2, eas2


def _sc_body(p_hbm, ea_hbm, eas_hbm, g_hbm, idx_hbm,
             idx_v, p_v, ea_v, eas_v, g_v, sem):
    wid = lax.axis_index("s") * NC + lax.axis_index("c")
    n_chunks = idx_hbm.shape[0] // NW
    pltpu.sync_copy(idx_hbm.at[pl.ds(wid * n_chunks, n_chunks)], idx_v)

    def chunk_body(j, _):
        idx_row = idx_v.at[j]
        pltpu.async_copy(p_hbm.at[idx_row], p_v, sem).wait()
        pltpu.async_copy(ea_hbm.at[idx_row], ea_v, sem).wait()
        pltpu.async_copy(eas_hbm.at[idx_row], eas_v, sem).wait()
        gbase = (wid * n_chunks + j) * CHUNK
        pltpu.sync_copy(g_hbm.at[pl.ds(gbase, CHUNK)], g_v)

        def row_body(i, _):
            for l in range(p_v.shape[1] // L):
                sl = pl.ds(l * L, L)
                p2, ea2, eas2 = _adam_vec(
                    p_v[i, sl], ea_v[i, sl], eas_v[i, sl], g_v[i, sl])
                p_v[i, sl] = p2
                ea_v[i, sl] = ea2
                eas_v[i, sl] = eas2
            return 0

        lax.fori_loop(0, CHUNK, row_body, 0)

        pltpu.async_copy(p_v, p_hbm.at[idx_row], sem).wait()
        pltpu.async_copy(ea_v, ea_hbm.at[idx_row], sem).wait()
        pltpu.async_copy(eas_v, eas_hbm.at[idx_row], sem).wait()
        return 0

    lax.fori_loop(0, n_chunks, chunk_body, 0)


def kernel(param, grad, exp_avg, exp_avg_sq, index):
    M, D = param.shape
    B = index.shape[0]
    assert B % (NW * CHUNK) == 0 and D % L == 0
    idx2d = index.astype(jnp.int32).reshape(B // CHUNK, CHUNK)

    p_ref = jax.new_ref(param)
    ea_ref = jax.new_ref(exp_avg)
    eas_ref = jax.new_ref(exp_avg_sq)

    mesh = plsc.VectorSubcoreMesh(
        core_axis_name="c", subcore_axis_name="s",
        num_cores=NC, num_subcores=NS)
    n_chunks = (B // CHUNK) // NW
    sc_update = pl.kernel(
        _sc_body,
        out_type=(),
        mesh=mesh,
        compiler_params=pltpu.CompilerParams(use_tc_tiling_on_sc=False),
        scratch_types=[
            pltpu.VMEM((n_chunks, CHUNK), jnp.int32),
            pltpu.VMEM((CHUNK, D), jnp.float32),
            pltpu.VMEM((CHUNK, D), jnp.float32),
            pltpu.VMEM((CHUNK, D), jnp.float32),
            pltpu.VMEM((CHUNK, D), jnp.float32),
            pltpu.SemaphoreType.DMA,
        ],
    )
    sc_update(p_ref, ea_ref, eas_ref, grad, idx2d)
    return p_ref[...], ea_ref[...], eas_ref[...]
